# baseline (speedup 1.0000x reference)
import functools

import jax
import jax.numpy as jnp
from jax import lax
from jax.experimental import pallas as pl
from jax.experimental.pallas import tpu as pltpu
from jax.experimental.pallas import tpu_sc as plsc

BIG_KERNEL = 31
SMALL_KERNEL = 5
C_OUT = 16
GROUP_IN = 4
NK = 7
C_IN = C_OUT * GROUP_IN * NK
L = 8192
B = 4
HALO = 16
ROWBUF = L + 2 * HALO
NCHUNK = L // 16

NUM_CORES = 2
NUM_SUBCORES = 16
NUM_WORKERS = NUM_CORES * NUM_SUBCORES
PAIRS_PER_WORKER = (B * C_OUT) // NUM_WORKERS


def _sc_body(
    x_hbm, tab_hbm, out1, out2, out3, rb0, rb1, idxbuf, acc1, acc2, acc3, sem0, sem1
):
    rbufs = (rb0, rb1)
    sems = (sem0, sem1)
    wid = lax.axis_index("s") * NUM_CORES + lax.axis_index("c")

    zeros16 = jnp.zeros((16,), jnp.float32)
    for rb in rbufs:
        rb[pl.ds(0, 16)] = zeros16
        rb[pl.ds(ROWBUF - 16, 16)] = zeros16
    iota16 = lax.iota(jnp.int32, 16)
    nci = GROUP_IN * NK

    for pp in range(PAIRS_PER_WORKER):
        p = wid * PAIRS_PER_WORKER + pp
        b = p // C_OUT
        o = lax.rem(p, C_OUT)

        ntab = 3 * nci * 16
        pltpu.sync_copy(tab_hbm.at[pl.ds(o * ntab, ntab)], idxbuf)

        @plsc.parallel_loop(0, L, 16, unroll=8)
        def _zero(i):
            sl = pl.ds(pl.multiple_of(i, 16), 16)
            acc1[sl] = zeros16
            acc2[sl] = zeros16
            acc3[sl] = zeros16

        def row_copy(ci, h):
            c = (ci // NK) * (C_OUT * NK) + o * NK + lax.rem(ci, NK)
            return pltpu.make_async_copy(
                x_hbm.at[pl.ds((b * C_IN + c) * L, L)],
                rbufs[h].at[pl.ds(HALO, L)],
                sems[h],
            )

        row_copy(0, 0).start()

        def chan2_body(cj, _):
            for h in range(2):
                ci = cj * 2 + h
                row_copy(ci, h).wait()

                @pl.when(ci + 1 < nci)
                def _():
                    row_copy(ci + 1, 1 - h).start()

                v1 = plsc.load_gather(idxbuf, [iota16 + ci * 16])
                v2 = plsc.load_gather(idxbuf, [iota16 + (nci + ci) * 16])
                v3 = plsc.load_gather(idxbuf, [iota16 + (2 * nci + ci) * 16])
                rbuf = rbufs[h]

                @plsc.parallel_loop(0, L, 16, unroll=16)
                def _chunk(i):
                    sl = pl.ds(pl.multiple_of(i, 16), 16)
                    plsc.addupdate(acc1.at[sl], plsc.load_gather(rbuf, [v1 + i]))
                    plsc.addupdate(acc2.at[sl], plsc.load_gather(rbuf, [v2 + i]))
                    plsc.addupdate(acc3.at[sl], plsc.load_gather(rbuf, [v3 + i]))

            return 0

        lax.fori_loop(0, nci // 2, chan2_body, 0)

        row = pl.ds((b * C_OUT + o) * L, L)
        pltpu.sync_copy(acc1, out1.at[row])
        pltpu.sync_copy(acc2, out2.at[row])
        pltpu.sync_copy(acc3, out3.at[row])


@jax.jit
def _run(x, shift_idx_1, shift_idx_2, shift_idx_3):
    center = (NK - 1) / 2.0
    real_pad = jnp.asarray(
        [int((center - i) * SMALL_KERNEL) for i in range(NK)], jnp.int32
    )
    gi = jnp.arange(GROUP_IN * NK, dtype=jnp.int32) // NK
    t = jnp.arange(GROUP_IN * NK, dtype=jnp.int32) % NK
    o_ids = jnp.arange(C_OUT, dtype=jnp.int32)
    c_table = gi[None, :] * (C_OUT * NK) + o_ids[:, None] * NK + t[None, :]

    def bases(idx):
        shifts = real_pad[idx % NK]
        base = HALO - shifts[c_table]
        return base[:, :, None] + jnp.arange(16, dtype=jnp.int32)[None, None, :]

    tab = jnp.stack(
        [bases(shift_idx_1), bases(shift_idx_2), bases(shift_idx_3)], axis=1
    ).reshape(C_OUT * 3 * GROUP_IN * NK * 16)

    mesh = plsc.VectorSubcoreMesh(
        core_axis_name="c",
        subcore_axis_name="s",
        num_cores=NUM_CORES,
        num_subcores=NUM_SUBCORES,
    )
    out_sd = jax.ShapeDtypeStruct((B * C_OUT * L,), jnp.float32)
    run = pl.kernel(
        _sc_body,
        out_type=(out_sd, out_sd, out_sd),
        mesh=mesh,
        scratch_types=[
            pltpu.VMEM((ROWBUF,), jnp.float32),
            pltpu.VMEM((ROWBUF,), jnp.float32),
            pltpu.VMEM((3 * GROUP_IN * NK * 16,), jnp.int32),
            pltpu.VMEM((L,), jnp.float32),
            pltpu.VMEM((L,), jnp.float32),
            pltpu.VMEM((L,), jnp.float32),
            pltpu.SemaphoreType.DMA,
            pltpu.SemaphoreType.DMA,
        ],
        compiler_params=pltpu.CompilerParams(needs_layout_passes=False),
    )
    o1, o2, o3 = run(x.reshape(-1), tab)
    shape = (B, C_OUT, L)
    return o1.reshape(shape), o2.reshape(shape), o3.reshape(shape)


def kernel(x, shift_idx_1, shift_idx_2, shift_idx_3):
    return _run(x, shift_idx_1, shift_idx_2, shift_idx_3)


# speedup vs baseline: 1.1578x; 1.1578x over previous
import functools

import jax
import jax.numpy as jnp
from jax import lax
from jax.experimental import pallas as pl
from jax.experimental.pallas import tpu as pltpu
from jax.experimental.pallas import tpu_sc as plsc

BIG_KERNEL = 31
SMALL_KERNEL = 5
C_OUT = 16
GROUP_IN = 4
NK = 7
C_IN = C_OUT * GROUP_IN * NK
L = 8192
B = 4
HALO = 16
ROWBUF = L + 2 * HALO
NCHUNK = L // 16

NUM_CORES = 2
NUM_SUBCORES = 16
NUM_WORKERS = NUM_CORES * NUM_SUBCORES
PAIRS_PER_WORKER = (B * C_OUT) // NUM_WORKERS


def _sc_body(
    x_hbm,
    tab_hbm,
    out1,
    out2,
    out3,
    rb0,
    rb1,
    rb2,
    rb3,
    idxbuf,
    acc1,
    acc2,
    acc3,
    sem0,
    sem1,
    sem2,
    sem3,
):
    rbufs = (rb0, rb1, rb2, rb3)
    sems = (sem0, sem1, sem2, sem3)
    wid = lax.axis_index("s") * NUM_CORES + lax.axis_index("c")

    zeros16 = jnp.zeros((16,), jnp.float32)
    for rb in rbufs:
        rb[pl.ds(0, 16)] = zeros16
        rb[pl.ds(ROWBUF - 16, 16)] = zeros16
    iota16 = lax.iota(jnp.int32, 16)
    nci = GROUP_IN * NK

    for pp in range(PAIRS_PER_WORKER):
        p = wid * PAIRS_PER_WORKER + pp
        b = p // C_OUT
        o = lax.rem(p, C_OUT)

        ntab = 3 * nci * 16
        pltpu.sync_copy(tab_hbm.at[pl.ds(o * ntab, ntab)], idxbuf)

        @plsc.parallel_loop(0, L, 16, unroll=8)
        def _zero(i):
            sl = pl.ds(pl.multiple_of(i, 16), 16)
            acc1[sl] = zeros16
            acc2[sl] = zeros16
            acc3[sl] = zeros16

        def row_copy(ci, h):
            c = (ci // NK) * (C_OUT * NK) + o * NK + lax.rem(ci, NK)
            return pltpu.make_async_copy(
                x_hbm.at[pl.ds((b * C_IN + c) * L, L)],
                rbufs[h].at[pl.ds(HALO, L)],
                sems[h],
            )

        row_copy(0, 0).start()
        row_copy(1, 1).start()

        def step_body(cj, _):
            for hh in range(2):
                s = cj * 2 + hh
                p = hh
                ca = s * 2
                cb = s * 2 + 1
                row_copy(ca, 2 * p).wait()
                row_copy(cb, 2 * p + 1).wait()

                @pl.when(s + 1 < nci // 2)
                def _():
                    row_copy(ca + 2, 2 * (1 - p)).start()
                    row_copy(cb + 2, 2 * (1 - p) + 1).start()

                v1a = plsc.load_gather(idxbuf, [iota16 + ca * 16])
                v2a = plsc.load_gather(idxbuf, [iota16 + (nci + ca) * 16])
                v3a = plsc.load_gather(idxbuf, [iota16 + (2 * nci + ca) * 16])
                v1b = plsc.load_gather(idxbuf, [iota16 + cb * 16])
                v2b = plsc.load_gather(idxbuf, [iota16 + (nci + cb) * 16])
                v3b = plsc.load_gather(idxbuf, [iota16 + (2 * nci + cb) * 16])
                rba = rbufs[2 * p]
                rbb = rbufs[2 * p + 1]

                @plsc.parallel_loop(0, L, 16, unroll=8)
                def _chunk(i):
                    sl = pl.ds(pl.multiple_of(i, 16), 16)
                    g1 = plsc.load_gather(rba, [v1a + i]) + plsc.load_gather(
                        rbb, [v1b + i]
                    )
                    g2 = plsc.load_gather(rba, [v2a + i]) + plsc.load_gather(
                        rbb, [v2b + i]
                    )
                    g3 = plsc.load_gather(rba, [v3a + i]) + plsc.load_gather(
                        rbb, [v3b + i]
                    )
                    plsc.addupdate(acc1.at[sl], g1)
                    plsc.addupdate(acc2.at[sl], g2)
                    plsc.addupdate(acc3.at[sl], g3)

            return 0

        lax.fori_loop(0, nci // 4, step_body, 0)

        row = pl.ds((b * C_OUT + o) * L, L)
        pltpu.sync_copy(acc1, out1.at[row])
        pltpu.sync_copy(acc2, out2.at[row])
        pltpu.sync_copy(acc3, out3.at[row])


@jax.jit
def _run(x, shift_idx_1, shift_idx_2, shift_idx_3):
    center = (NK - 1) / 2.0
    real_pad = jnp.asarray(
        [int((center - i) * SMALL_KERNEL) for i in range(NK)], jnp.int32
    )
    gi = jnp.arange(GROUP_IN * NK, dtype=jnp.int32) // NK
    t = jnp.arange(GROUP_IN * NK, dtype=jnp.int32) % NK
    o_ids = jnp.arange(C_OUT, dtype=jnp.int32)
    c_table = gi[None, :] * (C_OUT * NK) + o_ids[:, None] * NK + t[None, :]

    def bases(idx):
        shifts = real_pad[idx % NK]
        base = HALO - shifts[c_table]
        return base[:, :, None] + jnp.arange(16, dtype=jnp.int32)[None, None, :]

    tab = jnp.stack(
        [bases(shift_idx_1), bases(shift_idx_2), bases(shift_idx_3)], axis=1
    ).reshape(C_OUT * 3 * GROUP_IN * NK * 16)

    mesh = plsc.VectorSubcoreMesh(
        core_axis_name="c",
        subcore_axis_name="s",
        num_cores=NUM_CORES,
        num_subcores=NUM_SUBCORES,
    )
    out_sd = jax.ShapeDtypeStruct((B * C_OUT * L,), jnp.float32)
    run = pl.kernel(
        _sc_body,
        out_type=(out_sd, out_sd, out_sd),
        mesh=mesh,
        scratch_types=[
            pltpu.VMEM((ROWBUF,), jnp.float32),
            pltpu.VMEM((ROWBUF,), jnp.float32),
            pltpu.VMEM((ROWBUF,), jnp.float32),
            pltpu.VMEM((ROWBUF,), jnp.float32),
            pltpu.VMEM((3 * GROUP_IN * NK * 16,), jnp.int32),
            pltpu.VMEM((L,), jnp.float32),
            pltpu.VMEM((L,), jnp.float32),
            pltpu.VMEM((L,), jnp.float32),
            pltpu.SemaphoreType.DMA,
            pltpu.SemaphoreType.DMA,
            pltpu.SemaphoreType.DMA,
            pltpu.SemaphoreType.DMA,
        ],
        compiler_params=pltpu.CompilerParams(needs_layout_passes=False),
    )
    o1, o2, o3 = run(x.reshape(-1), tab)
    shape = (B, C_OUT, L)
    return o1.reshape(shape), o2.reshape(shape), o3.reshape(shape)


def kernel(x, shift_idx_1, shift_idx_2, shift_idx_3):
    return _run(x, shift_idx_1, shift_idx_2, shift_idx_3)


# speedup vs baseline: 1.1847x; 1.0232x over previous
import functools

import jax
import jax.numpy as jnp
from jax import lax
from jax.experimental import pallas as pl
from jax.experimental.pallas import tpu as pltpu
from jax.experimental.pallas import tpu_sc as plsc

BIG_KERNEL = 31
SMALL_KERNEL = 5
C_OUT = 16
GROUP_IN = 4
NK = 7
C_IN = C_OUT * GROUP_IN * NK
L = 8192
B = 4
HALO = 16
ROWBUF = L + 2 * HALO
NCHUNK = L // 16

NUM_CORES = 2
NUM_SUBCORES = 16
NUM_WORKERS = NUM_CORES * NUM_SUBCORES
PAIRS_PER_WORKER = (B * C_OUT) // NUM_WORKERS


def _sc_body(
    x_hbm,
    tab_hbm,
    out1,
    out2,
    out3,
    rb0,
    rb1,
    rb2,
    rb3,
    idxbuf,
    acc1,
    acc2,
    acc3,
    sem0,
    sem1,
    sem2,
    sem3,
    tsem,
    wsem,
):
    rbufs = (rb0, rb1, rb2, rb3)
    sems = (sem0, sem1, sem2, sem3)
    wid = lax.axis_index("s") * NUM_CORES + lax.axis_index("c")

    zeros16 = jnp.zeros((16,), jnp.float32)
    for rb in rbufs:
        rb[pl.ds(0, 16)] = zeros16
        rb[pl.ds(ROWBUF - 16, 16)] = zeros16
    iota16 = lax.iota(jnp.int32, 16)
    nci = GROUP_IN * NK

    for pp in range(PAIRS_PER_WORKER):
        p = wid * PAIRS_PER_WORKER + pp
        b = p // C_OUT
        o = lax.rem(p, C_OUT)

        ntab = 3 * nci * 16
        tab_copy = pltpu.make_async_copy(
            tab_hbm.at[pl.ds(o * ntab, ntab)], idxbuf, tsem
        )
        tab_copy.start()

        def row_copy(ci, h):
            c = (ci // NK) * (C_OUT * NK) + o * NK + lax.rem(ci, NK)
            return pltpu.make_async_copy(
                x_hbm.at[pl.ds((b * C_IN + c) * L, L)],
                rbufs[h].at[pl.ds(HALO, L)],
                sems[h],
            )

        row_copy(0, 0).start()
        row_copy(1, 1).start()

        if pp > 0:
            for cp in prev_wb:
                cp.wait()

        @plsc.parallel_loop(0, L, 16, unroll=8)
        def _zero(i):
            sl = pl.ds(pl.multiple_of(i, 16), 16)
            acc1[sl] = zeros16
            acc2[sl] = zeros16
            acc3[sl] = zeros16

        tab_copy.wait()

        def step_body(cj, _):
            for hh in range(2):
                s = cj * 2 + hh
                p = hh
                ca = s * 2
                cb = s * 2 + 1
                row_copy(ca, 2 * p).wait()
                row_copy(cb, 2 * p + 1).wait()

                @pl.when(s + 1 < nci // 2)
                def _():
                    row_copy(ca + 2, 2 * (1 - p)).start()
                    row_copy(cb + 2, 2 * (1 - p) + 1).start()

                v1a = plsc.load_gather(idxbuf, [iota16 + ca * 16])
                v2a = plsc.load_gather(idxbuf, [iota16 + (nci + ca) * 16])
                v3a = plsc.load_gather(idxbuf, [iota16 + (2 * nci + ca) * 16])
                v1b = plsc.load_gather(idxbuf, [iota16 + cb * 16])
                v2b = plsc.load_gather(idxbuf, [iota16 + (nci + cb) * 16])
                v3b = plsc.load_gather(idxbuf, [iota16 + (2 * nci + cb) * 16])
                rba = rbufs[2 * p]
                rbb = rbufs[2 * p + 1]

                @plsc.parallel_loop(0, L, 16, unroll=8)
                def _chunk(i):
                    sl = pl.ds(pl.multiple_of(i, 16), 16)
                    g1 = plsc.load_gather(rba, [v1a + i]) + plsc.load_gather(
                        rbb, [v1b + i]
                    )
                    g2 = plsc.load_gather(rba, [v2a + i]) + plsc.load_gather(
                        rbb, [v2b + i]
                    )
                    g3 = plsc.load_gather(rba, [v3a + i]) + plsc.load_gather(
                        rbb, [v3b + i]
                    )
                    plsc.addupdate(acc1.at[sl], g1)
                    plsc.addupdate(acc2.at[sl], g2)
                    plsc.addupdate(acc3.at[sl], g3)

            return 0

        lax.fori_loop(0, nci // 4, step_body, 0)

        row = pl.ds((b * C_OUT + o) * L, L)
        prev_wb = (
            pltpu.make_async_copy(acc1, out1.at[row], wsem),
            pltpu.make_async_copy(acc2, out2.at[row], wsem),
            pltpu.make_async_copy(acc3, out3.at[row], wsem),
        )
        for cp in prev_wb:
            cp.start()

    for cp in prev_wb:
        cp.wait()


@jax.jit
def _run(x, shift_idx_1, shift_idx_2, shift_idx_3):
    center = (NK - 1) / 2.0
    real_pad = jnp.asarray(
        [int((center - i) * SMALL_KERNEL) for i in range(NK)], jnp.int32
    )
    gi = jnp.arange(GROUP_IN * NK, dtype=jnp.int32) // NK
    t = jnp.arange(GROUP_IN * NK, dtype=jnp.int32) % NK
    o_ids = jnp.arange(C_OUT, dtype=jnp.int32)
    c_table = gi[None, :] * (C_OUT * NK) + o_ids[:, None] * NK + t[None, :]

    def bases(idx):
        shifts = real_pad[idx % NK]
        base = HALO - shifts[c_table]
        return base[:, :, None] + jnp.arange(16, dtype=jnp.int32)[None, None, :]

    tab = jnp.stack(
        [bases(shift_idx_1), bases(shift_idx_2), bases(shift_idx_3)], axis=1
    ).reshape(C_OUT * 3 * GROUP_IN * NK * 16)

    mesh = plsc.VectorSubcoreMesh(
        core_axis_name="c",
        subcore_axis_name="s",
        num_cores=NUM_CORES,
        num_subcores=NUM_SUBCORES,
    )
    out_sd = jax.ShapeDtypeStruct((B * C_OUT * L,), jnp.float32)
    run = pl.kernel(
        _sc_body,
        out_type=(out_sd, out_sd, out_sd),
        mesh=mesh,
        scratch_types=[
            pltpu.VMEM((ROWBUF,), jnp.float32),
            pltpu.VMEM((ROWBUF,), jnp.float32),
            pltpu.VMEM((ROWBUF,), jnp.float32),
            pltpu.VMEM((ROWBUF,), jnp.float32),
            pltpu.VMEM((3 * GROUP_IN * NK * 16,), jnp.int32),
            pltpu.VMEM((L,), jnp.float32),
            pltpu.VMEM((L,), jnp.float32),
            pltpu.VMEM((L,), jnp.float32),
            pltpu.SemaphoreType.DMA,
            pltpu.SemaphoreType.DMA,
            pltpu.SemaphoreType.DMA,
            pltpu.SemaphoreType.DMA,
            pltpu.SemaphoreType.DMA,
            pltpu.SemaphoreType.DMA,
        ],
        compiler_params=pltpu.CompilerParams(needs_layout_passes=False),
    )
    o1, o2, o3 = run(x.reshape(-1), tab)
    shape = (B, C_OUT, L)
    return o1.reshape(shape), o2.reshape(shape), o3.reshape(shape)


def kernel(x, shift_idx_1, shift_idx_2, shift_idx_3):
    return _run(x, shift_idx_1, shift_idx_2, shift_idx_3)
